# baseline (speedup 1.0000x reference)
import functools

import jax
import jax.numpy as jnp
from jax import lax
from jax.experimental import pallas as pl
from jax.experimental.pallas import tpu as pltpu
from jax.experimental.pallas import tpu_sc as plsc


def kernel(inputs, word_table, pos_table):
    B, S = inputs.shape
    V, D = word_table.shape
    info = plsc.get_sparse_core_info()
    NC, NS, L = info.num_cores, info.num_subcores, info.num_lanes
    assert D % NC == 0 and B % 128 == 0
    d_per_c = D // NC
    BR = B // 128
    ns_lo, hi = divmod(S, NS)
    ns_hi = ns_lo + 1
    SMAX = ns_hi * B
    VSL = (V // NS) // 128 * 128
    VREM = V - NS * VSL
    VREM_AL = VREM // 128 * 128
    VTAIL = VREM - VREM_AL

    idx_sm = inputs.T.reshape(S * B)
    idx_pad = jnp.pad(idx_sm, (0, SMAX * NS - S * B))
    pos_pad = jnp.pad(pos_table.reshape(S * D), (0, ns_hi * D * NS - S * D))
    tableT = word_table.T
    tail1d = word_table[V - VTAIL:, :].T.reshape(D * VTAIL) if VTAIL else None

    mesh = plsc.VectorSubcoreMesh(core_axis_name="c", subcore_axis_name="s")

    def make_part(d_lo, d_n):
      @functools.partial(
        pl.kernel,
        out_type=jax.ShapeDtypeStruct((S, NC, d_n, BR, 128), jnp.float32),
        mesh=mesh,
        scratch_types=[
            pltpu.VMEM_SHARED((V,), jnp.float32),
            pltpu.VMEM_SHARED((V,), jnp.float32),
            pltpu.VMEM((B,), jnp.float32),
            pltpu.VMEM((1, BR, 128), jnp.float32),
            pltpu.VMEM((B,), jnp.int32),
            pltpu.VMEM((ns_hi * D,), jnp.float32),
            pltpu.VMEM((max(VTAIL, L),), jnp.float32),
            pltpu.SemaphoreType.DMA,
            pltpu.SemaphoreType.DMA,
        ],
        compiler_params=pltpu.CompilerParams(
            use_tc_tiling_on_sc=True, needs_layout_passes=False),
    )
      def emb_kernel(idx_hbm, tT, pos_hbm, tail_hbm, out_hbm,
                     slab_a, slab_b, vals, stage, idx_c, pos_v,
                     tvmem, ssem, gsem):
        c = lax.axis_index("c")
        t = lax.axis_index("s")
        s0 = jnp.where(t < hi, ns_hi * t, ns_lo * t + hi)
        ns = jnp.where(t < hi, ns_hi, ns_lo)
        pltpu.sync_copy(pos_hbm.at[pl.ds(s0 * D, ns_hi * D)], pos_v)

        def slab_piece(di, slab):
            for cc in range(NC):
                @pl.when(c == cc)
                def _(cc=cc):
                    d = cc * d_per_c + di
                    for tt in range(NS):
                        @pl.when(t == tt)
                        def _(tt=tt, d=d):
                            pltpu.async_copy(
                                tT.at[d, pl.ds(tt * VSL, VSL)],
                                slab.at[pl.ds(tt * VSL, VSL)], ssem)
                    if VREM_AL:
                        @pl.when(t == NS - 1)
                        def _(d=d):
                            pltpu.async_copy(
                                tT.at[d, pl.ds(NS * VSL, VREM_AL)],
                                slab.at[pl.ds(NS * VSL, VREM_AL)], ssem)
                    if VTAIL:
                        @pl.when(t == NS - 1)
                        def _(d=d):
                            pltpu.async_copy(
                                tail_hbm.at[pl.ds(d * VTAIL, VTAIL)],
                                tvmem.at[pl.ds(0, VTAIL)], ssem)

        def slab_wait(slab):
            pltpu.make_async_copy(
                tT.at[0, pl.ds(0, VSL)],
                slab.at[pl.ds(pl.multiple_of(t * VSL, 128), VSL)],
                ssem).wait()
            if VREM_AL:
                @pl.when(t == NS - 1)
                def _():
                    pltpu.make_async_copy(
                        tT.at[0, pl.ds(0, VREM_AL)],
                        slab.at[pl.ds(NS * VSL, VREM_AL)], ssem).wait()
            if VTAIL:
                @pl.when(t == NS - 1)
                def _():
                    pltpu.make_async_copy(
                        tail_hbm.at[pl.ds(0, VTAIL)],
                        tvmem.at[pl.ds(0, VTAIL)], ssem).wait()
                    pltpu.sync_copy(tvmem.at[pl.ds(0, VTAIL)],
                                    slab.at[pl.ds(V - VTAIL, VTAIL)])

        slab_piece(d_lo, slab_a)

        for dii in range(d_n):
            di = d_lo + dii
            slab = slab_a if dii % 2 == 0 else slab_b
            nslab = slab_b if dii % 2 == 0 else slab_a
            slab_wait(slab)
            plsc.subcore_barrier()

            if di + 1 < d_lo + d_n:
                slab_piece(di + 1, nslab)

            def do_seq(si, di=di):
                pv = plsc.load_gather(
                    pos_v,
                    [jnp.broadcast_to(si * D + c * d_per_c + di, (L,))])

                def jbody(j, c3):
                    sl = pl.ds(j * L, L)
                    jr = j // (128 // L)
                    jc = (j % (128 // L)) * L
                    stage[0, jr, pl.ds(jc, L)] = vals[sl] + pv
                    return c3

                lax.fori_loop(0, B // L, jbody, 0)

            def sbody(si, c2, di=di):
                pltpu.sync_copy(
                    idx_hbm.at[pl.ds(s0 * B + si * B, B)], idx_c)
                pltpu.async_copy(slab.at[idx_c], vals, gsem).wait()
                do_seq(si, di)
                for cc in range(NC):
                    @pl.when(c == cc)
                    def _(cc=cc, di=di):
                        pltpu.sync_copy(
                            stage,
                            out_hbm.at[pl.ds(s0 + si, 1), cc, dii, :, :])
                return c2

            lax.fori_loop(0, ns, sbody, 0)

      return emb_kernel

    half = d_per_c // 2
    out_a = make_part(0, half)(idx_pad, tableT, pos_pad, tail1d)
    out_b = make_part(half, d_per_c - half)(idx_pad, tableT, pos_pad, tail1d)
    out5 = jnp.concatenate([out_a, out_b], axis=2)
    return jnp.transpose(out5.reshape(S, D, B), (2, 0, 1))


# speedup vs baseline: 1.5761x; 1.5761x over previous
import functools

import jax
import jax.numpy as jnp
from jax import lax
from jax.experimental import pallas as pl
from jax.experimental.pallas import tpu as pltpu
from jax.experimental.pallas import tpu_sc as plsc


def kernel(inputs, word_table, pos_table):
    B, S = inputs.shape
    V, D = word_table.shape
    info = plsc.get_sparse_core_info()
    NC, NS, L = info.num_cores, info.num_subcores, info.num_lanes
    assert D % NC == 0 and B % 128 == 0
    d_per_c = D // NC
    BR = B // 128
    ns_lo, hi = divmod(S, NS)
    ns_hi = ns_lo + 1
    SMAX = ns_hi * B
    VSL = (V // NS) // 128 * 128
    VREM = V - NS * VSL
    VREM_AL = VREM // 128 * 128
    VTAIL = VREM - VREM_AL

    idx_sm = inputs.T.reshape(S * B)
    idx_pad = jnp.pad(idx_sm, (0, SMAX * NS - S * B))
    pos_pad = jnp.pad(pos_table.reshape(S * D), (0, ns_hi * D * NS - S * D))
    tableT = word_table.T
    tail1d = word_table[V - VTAIL:, :].T.reshape(D * VTAIL) if VTAIL else None

    mesh = plsc.VectorSubcoreMesh(core_axis_name="c", subcore_axis_name="s")

    def make_part(d_lo, d_n):
      @functools.partial(
        pl.kernel,
        out_type=jax.ShapeDtypeStruct((S, NC, d_n, BR, 128), jnp.float32),
        mesh=mesh,
        scratch_types=[
            pltpu.VMEM_SHARED((V,), jnp.float32),
            pltpu.VMEM((SMAX,), jnp.int32),
            pltpu.VMEM((SMAX,), jnp.float32),
            pltpu.VMEM((ns_hi, BR, 128), jnp.float32),
            pltpu.VMEM((ns_hi * D,), jnp.float32),
            pltpu.VMEM((max(VTAIL, L),), jnp.float32),
            pltpu.SemaphoreType.DMA,
            pltpu.SemaphoreType.DMA,
        ],
        compiler_params=pltpu.CompilerParams(
            use_tc_tiling_on_sc=True, needs_layout_passes=False),
    )
      def emb_kernel(idx_hbm, tT, pos_hbm, tail_hbm, out_hbm,
                     slab, idx_v, vals, stage, pos_v, tvmem, ssem, gsem):
        c = lax.axis_index("c")
        t = lax.axis_index("s")
        s0 = jnp.where(t < hi, ns_hi * t, ns_lo * t + hi)
        ns = jnp.where(t < hi, ns_hi, ns_lo)
        pltpu.sync_copy(idx_hbm.at[pl.ds(s0 * B, SMAX)], idx_v)
        pltpu.sync_copy(pos_hbm.at[pl.ds(s0 * D, ns_hi * D)], pos_v)

        def slab_piece(di):
            for cc in range(NC):
                @pl.when(c == cc)
                def _(cc=cc):
                    d = cc * d_per_c + di
                    for tt in range(NS):
                        @pl.when(t == tt)
                        def _(tt=tt, d=d):
                            pltpu.async_copy(
                                tT.at[d, pl.ds(tt * VSL, VSL)],
                                slab.at[pl.ds(tt * VSL, VSL)], ssem)
                    if VREM_AL:
                        @pl.when(t == NS - 1)
                        def _(d=d):
                            pltpu.async_copy(
                                tT.at[d, pl.ds(NS * VSL, VREM_AL)],
                                slab.at[pl.ds(NS * VSL, VREM_AL)], ssem)
                    if VTAIL:
                        @pl.when(t == NS - 1)
                        def _(d=d):
                            pltpu.async_copy(
                                tail_hbm.at[pl.ds(d * VTAIL, VTAIL)],
                                tvmem.at[pl.ds(0, VTAIL)], ssem)

        def slab_wait():
            pltpu.make_async_copy(
                tT.at[0, pl.ds(0, VSL)],
                slab.at[pl.ds(pl.multiple_of(t * VSL, 128), VSL)],
                ssem).wait()
            if VREM_AL:
                @pl.when(t == NS - 1)
                def _():
                    pltpu.make_async_copy(
                        tT.at[0, pl.ds(0, VREM_AL)],
                        slab.at[pl.ds(NS * VSL, VREM_AL)], ssem).wait()
            if VTAIL:
                @pl.when(t == NS - 1)
                def _():
                    pltpu.make_async_copy(
                        tail_hbm.at[pl.ds(0, VTAIL)],
                        tvmem.at[pl.ds(0, VTAIL)], ssem).wait()
                    pltpu.sync_copy(tvmem.at[pl.ds(0, VTAIL)],
                                    slab.at[pl.ds(V - VTAIL, VTAIL)])

        slab_piece(d_lo)

        for dii in range(d_n):
            di = d_lo + dii
            slab_wait()
            plsc.subcore_barrier()
            pltpu.async_copy(slab.at[idx_v], vals, gsem).wait()
            plsc.subcore_barrier()

            if di + 1 < d_lo + d_n:
                slab_piece(di + 1)

            def sbody(si, c2, di=di):
                pv = plsc.load_gather(
                    pos_v,
                    [jnp.broadcast_to(si * D + c * d_per_c + di, (L,))])

                def jbody(j, c3):
                    sl = pl.ds(si * B + j * L, L)
                    jr = j // (128 // L)
                    jc = (j % (128 // L)) * L
                    stage[si, jr, pl.ds(jc, L)] = vals[sl] + pv
                    return c3

                lax.fori_loop(0, B // L, jbody, 0)
                return c2

            lax.fori_loop(0, ns, sbody, 0)

            for cc in range(NC):
                @pl.when(jnp.logical_and(c == cc, t < hi))
                def _(cc=cc, di=di):
                    pltpu.sync_copy(
                        stage.at[pl.ds(0, ns_hi)],
                        out_hbm.at[pl.ds(s0, ns_hi), cc, dii, :, :])

                @pl.when(jnp.logical_and(c == cc, t >= hi))
                def _(cc=cc, di=di):
                    pltpu.sync_copy(
                        stage.at[pl.ds(0, ns_lo)],
                        out_hbm.at[pl.ds(s0, ns_lo), cc, dii, :, :])

      return emb_kernel

    half = d_per_c // 2
    out_a = make_part(0, half)(idx_pad, tableT, pos_pad, tail1d)
    out_b = make_part(half, d_per_c - half)(idx_pad, tableT, pos_pad, tail1d)
    out5 = jnp.concatenate([out_a, out_b], axis=2)
    return jnp.transpose(out5.reshape(S, D, B), (2, 0, 1))
